# baseline (speedup 1.0000x reference)
import functools

import jax
import jax.numpy as jnp
from jax import lax
from jax.experimental import pallas as pl
from jax.experimental.pallas import tpu as pltpu
from jax.experimental.pallas import tpu_sc as plsc

D = 32
V = 100000
B = 16384
NT = 4
NC, NS = 2, 16
NW = NC * NS
BPW = B // NW

TBLK = 4096
NQ = 4
TGRID = 7
V2 = TGRID * TBLK
L = 16


def _tr_body(*refs):
    ws, outs = refs[: NT * NQ], refs[NT * NQ :]
    for c in range(NT):
        stack = jnp.concatenate([ws[c * NQ + q][...] for q in range(NQ)], axis=0)
        outs[c][...] = jnp.transpose(stack)


_tc_transpose = pl.pallas_call(
    _tr_body,
    grid=(TGRID,),
    in_specs=[
        pl.BlockSpec(
            (D, TBLK),
            functools.partial(
                lambda q, i: (0, jnp.minimum(q * TGRID + i, (V - 1) // TBLK)), q
            ),
        )
        for _ in range(NT)
        for q in range(NQ)
    ],
    out_specs=[
        pl.BlockSpec((TBLK, NQ * D), lambda i: (i, 0)) for _ in range(NT)
    ],
    out_shape=[
        jax.ShapeDtypeStruct((V2, NQ * D), jnp.float32) for _ in range(NT)
    ],
)


@functools.partial(
    pl.kernel,
    mesh=plsc.VectorSubcoreMesh(core_axis_name="c", subcore_axis_name="s"),
    out_type=jax.ShapeDtypeStruct((B, NT * D), jnp.float32),
    scratch_types=[
        pltpu.VMEM((NT * BPW,), jnp.int32),
        pltpu.VMEM((NT, BPW, D), jnp.float32),
        pltpu.SemaphoreType.DMA,
        pltpu.SemaphoreType.DMA,
    ],
    compiler_params=pltpu.CompilerParams(use_tc_tiling_on_sc=False),
)
def _gather4(xt, w0, w1, w2, w3, out, idx_v, rows_v, sem, wsem):
    wid = lax.axis_index("s") * NC + lax.axis_index("c")
    base = wid * BPW
    for c in range(NT):
        pltpu.sync_copy(
            xt.at[c, pl.ds(base, BPW)], idx_v.at[pl.ds(c * BPW, BPW)]
        )

    tables = (w0, w1, w2, w3)
    copies = []
    for c in range(NT):
        copies.append(
            pltpu.async_copy(
                tables[c].at[idx_v.at[pl.ds(c * BPW, BPW)]], rows_v.at[c], sem
            )
        )
    writes = []
    for c in range(NT):
        copies[c].wait()
        writes.append(
            pltpu.async_copy(
                rows_v.at[c], out.at[pl.ds(base, BPW), pl.ds(c * D, D)], wsem
            )
        )
    for c in range(NT):
        writes[c].wait()


def kernel(x, W0, W1, W2, W3):
    lin = _tc_transpose(
        jnp.transpose(W0), jnp.transpose(W0), jnp.transpose(W0), jnp.transpose(W0),
        jnp.transpose(W1), jnp.transpose(W1), jnp.transpose(W1), jnp.transpose(W1),
        jnp.transpose(W2), jnp.transpose(W2), jnp.transpose(W2), jnp.transpose(W2),
        jnp.transpose(W3), jnp.transpose(W3), jnp.transpose(W3), jnp.transpose(W3),
    )
    tabs = [l.reshape(NQ * V2, D) for l in lin]
    n = x.astype(jnp.int32)
    m = NQ * (n % V2) + n // V2
    return _gather4(jnp.transpose(m), *tabs)


# speedup vs baseline: 1.0333x; 1.0333x over previous
import functools

import jax
import jax.numpy as jnp
from jax import lax
from jax.experimental import pallas as pl
from jax.experimental.pallas import tpu as pltpu
from jax.experimental.pallas import tpu_sc as plsc

D = 32
V = 100000
B = 16384
NT = 4
NC, NS = 2, 16
NW = NC * NS
BPW = B // NW

TBLK = 3584
NQ = 4
TGRID = 7
V2 = TGRID * TBLK
L = 16


def _tr_body(*refs):
    ws, outs = refs[: NT * NQ], refs[NT * NQ :]
    for c in range(NT):
        stack = jnp.concatenate([ws[c * NQ + q][...] for q in range(NQ)], axis=0)
        outs[c][...] = jnp.transpose(stack)


_tc_transpose = pl.pallas_call(
    _tr_body,
    grid=(TGRID,),
    in_specs=[
        pl.BlockSpec(
            (D, TBLK),
            functools.partial(
                lambda q, i: (0, jnp.minimum(q * TGRID + i, (V - 1) // TBLK)), q
            ),
        )
        for _ in range(NT)
        for q in range(NQ)
    ],
    out_specs=[
        pl.BlockSpec((TBLK, NQ * D), lambda i: (i, 0)) for _ in range(NT)
    ],
    out_shape=[
        jax.ShapeDtypeStruct((V2, NQ * D), jnp.float32) for _ in range(NT)
    ],
)


@functools.partial(
    pl.kernel,
    mesh=plsc.VectorSubcoreMesh(core_axis_name="c", subcore_axis_name="s"),
    out_type=jax.ShapeDtypeStruct((B, NT * D), jnp.float32),
    scratch_types=[
        pltpu.VMEM((NT * BPW,), jnp.int32),
        pltpu.VMEM((NT, BPW, D), jnp.float32),
        pltpu.SemaphoreType.DMA,
        pltpu.SemaphoreType.DMA,
    ],
    compiler_params=pltpu.CompilerParams(use_tc_tiling_on_sc=False),
)
def _gather4(xt, w0, w1, w2, w3, out, idx_v, rows_v, sem, wsem):
    wid = lax.axis_index("s") * NC + lax.axis_index("c")
    base = wid * BPW
    for c in range(NT):
        pltpu.sync_copy(
            xt.at[c, pl.ds(base, BPW)], idx_v.at[pl.ds(c * BPW, BPW)]
        )

    tables = (w0, w1, w2, w3)
    copies = []
    for c in range(NT):
        copies.append(
            pltpu.async_copy(
                tables[c].at[idx_v.at[pl.ds(c * BPW, BPW)]], rows_v.at[c], sem
            )
        )
    writes = []
    for c in range(NT):
        copies[c].wait()
        writes.append(
            pltpu.async_copy(
                rows_v.at[c], out.at[pl.ds(base, BPW), pl.ds(c * D, D)], wsem
            )
        )
    for c in range(NT):
        writes[c].wait()


def kernel(x, W0, W1, W2, W3):
    lin = _tc_transpose(
        jnp.transpose(W0), jnp.transpose(W0), jnp.transpose(W0), jnp.transpose(W0),
        jnp.transpose(W1), jnp.transpose(W1), jnp.transpose(W1), jnp.transpose(W1),
        jnp.transpose(W2), jnp.transpose(W2), jnp.transpose(W2), jnp.transpose(W2),
        jnp.transpose(W3), jnp.transpose(W3), jnp.transpose(W3), jnp.transpose(W3),
    )
    tabs = [l.reshape(NQ * V2, D) for l in lin]
    n = x.astype(jnp.int32)
    m = NQ * (n % V2) + n // V2
    return _gather4(jnp.transpose(m), *tabs)


# speedup vs baseline: 1.0585x; 1.0244x over previous
import functools

import jax
import jax.numpy as jnp
from jax import lax
from jax.experimental import pallas as pl
from jax.experimental.pallas import tpu as pltpu
from jax.experimental.pallas import tpu_sc as plsc

D = 32
V = 100000
B = 16384
NT = 4
NC, NS = 2, 16
NW = NC * NS
BPW = B // NW

TBLK = 3584
NQ = 4
TGRID = 7
V2 = TGRID * TBLK
L = 16


def _tr_body(*refs):
    ws, outs = refs[: NT * NQ], refs[NT * NQ :]
    for c in range(NT):
        stack = jnp.concatenate([ws[c * NQ + q][...] for q in range(NQ)], axis=0)
        outs[c][...] = jnp.transpose(stack)


_tc_transpose = pl.pallas_call(
    _tr_body,
    grid=(TGRID,),
    in_specs=[
        pl.BlockSpec(
            (D, TBLK),
            functools.partial(
                lambda q, i: (0, jnp.minimum(q * TGRID + i, (V - 1) // TBLK)), q
            ),
        )
        for _ in range(NT)
        for q in range(NQ)
    ],
    out_specs=[
        pl.BlockSpec((TBLK, NQ * D), lambda i: (i, 0)) for _ in range(NT)
    ],
    out_shape=[
        jax.ShapeDtypeStruct((V2, NQ * D), jnp.float32) for _ in range(NT)
    ],
)


@functools.partial(
    pl.kernel,
    mesh=plsc.VectorSubcoreMesh(core_axis_name="c", subcore_axis_name="s"),
    out_type=jax.ShapeDtypeStruct((B, NT * D), jnp.float32),
    scratch_types=[
        pltpu.VMEM((NT * BPW,), jnp.int32),
        pltpu.VMEM((NT, BPW, D), jnp.float32),
        pltpu.SemaphoreType.DMA,
        pltpu.SemaphoreType.DMA,
    ],
    compiler_params=pltpu.CompilerParams(use_tc_tiling_on_sc=False),
)
def _gather4(xt, w0, w1, w2, w3, out, idx_v, rows_v, sem, wsem):
    wid = lax.axis_index("s") * NC + lax.axis_index("c")
    base = wid * BPW
    pltpu.sync_copy(xt.at[pl.ds(base * NT, NT * BPW)], idx_v)

    tables = (w0, w1, w2, w3)
    copies = []
    for c in range(NT):
        copies.append(
            pltpu.async_copy(
                tables[c].at[idx_v.at[pl.ds(c * BPW, BPW)]], rows_v.at[c], sem
            )
        )
    writes = []
    for c in range(NT):
        copies[c].wait()
        writes.append(
            pltpu.async_copy(
                rows_v.at[c], out.at[pl.ds(base, BPW), pl.ds(c * D, D)], wsem
            )
        )
    for c in range(NT):
        writes[c].wait()


def kernel(x, W0, W1, W2, W3):
    lin = _tc_transpose(
        jnp.transpose(W0), jnp.transpose(W0), jnp.transpose(W0), jnp.transpose(W0),
        jnp.transpose(W1), jnp.transpose(W1), jnp.transpose(W1), jnp.transpose(W1),
        jnp.transpose(W2), jnp.transpose(W2), jnp.transpose(W2), jnp.transpose(W2),
        jnp.transpose(W3), jnp.transpose(W3), jnp.transpose(W3), jnp.transpose(W3),
    )
    tabs = [l.reshape(NQ * V2, D) for l in lin]
    n = x.astype(jnp.int32)
    m = NQ * (n % V2) + n // V2
    xt = jnp.transpose(m.reshape(NW, BPW, NT), (0, 2, 1)).reshape(-1)
    return _gather4(xt, *tabs)
